# baseline (speedup 1.0000x reference)
import functools

import jax
import jax.numpy as jnp
from jax import lax
from jax.experimental import pallas as pl
from jax.experimental.pallas import tpu as pltpu
from jax.experimental.pallas import tpu_sc as plsc

K = 64
ROWS = 128
COLS = 32768
NVEC = COLS // 16
NC = 2
NS = 16
NW = NC * NS
ROWS_PER_W = ROWS // NW

_INT_MIN = -(2 ** 31)


def _lanes():
    return jnp.arange(16, dtype=jnp.int32)


def _splat_to_scalar(v):
    return lax.reduce_max(v, axes=(0,))


def _extract(v, idx):
    return lax.reduce_max(
        jnp.where(_lanes() == idx, v, jnp.int32(_INT_MIN)), axes=(0,)
    )


def _popcount_scalar(mask):
    return _splat_to_scalar(plsc.all_reduce_population_count(mask))


def _monotone(v):
    b = lax.bitcast_convert_type(v, jnp.int32)
    return jnp.where(
        b >= 0, b, jnp.bitwise_xor(jnp.bitwise_not(b), jnp.int32(_INT_MIN))
    )


def _zero_hist(hist):
    def body(j, c):
        hist[pl.ds(pl.multiple_of(j * 16, 16), 16)] = jnp.zeros(16, jnp.int32)
        return c

    lax.fori_loop(0, 256, body, 0)


def _merge_hist(hist, tot):
    def body(j, c):
        off = pl.multiple_of(j * 16, 16)
        acc = jnp.zeros(16, jnp.int32)
        for l in range(16):
            acc = acc + hist[pl.ds(l * 256 + off, 16)]
        tot[pl.ds(off, 16)] = acc
        return c

    lax.fori_loop(0, 16, body, 0)


def _find_boundary(tot, kp):
    lanes = _lanes()
    s = jnp.zeros(16, jnp.int32)
    for j in range(16):
        sj = lax.reduce_sum(tot[pl.ds(j * 16, 16)], axes=(0,))
        s = jnp.where(lanes == j, sj, s)
    rev_s = lax.rev(s, (0,))
    cs_s = plsc.cumsum(rev_s)
    hit_s = cs_s >= kp
    lc = jnp.int32(16) - _popcount_scalar(hit_s)
    jc = jnp.int32(15) - lc
    above_chunks = _extract(cs_s, lc) - _extract(rev_s, lc)

    chunk = tot[pl.ds(pl.multiple_of(jc * 16, 16), 16)]
    rchunk = lax.rev(chunk, (0,))
    cs2 = above_chunks + plsc.cumsum(rchunk)
    hit2 = cs2 >= kp
    l2 = jnp.int32(16) - _popcount_scalar(hit2)
    dstar = jc * 16 + jnp.int32(15) - l2
    cnt_gt = _extract(cs2, l2) - _extract(rchunk, l2)
    return dstar, kp - cnt_gt


def _sc_body(x_hbm, out_hbm, rowbuf, cand, hist, tot):
    wid = lax.axis_index("s") * NC + lax.axis_index("c")
    lanes = _lanes()
    ones = jnp.ones(16, jnp.int32)

    def do_row(r, carry):
        row = wid * ROWS_PER_W + r
        pltpu.sync_copy(x_hbm.at[row], rowbuf)

        _zero_hist(hist)

        def hist0_body(i, c):
            v = rowbuf[pl.ds(pl.multiple_of(i * 16, 16), 16)]
            m = _monotone(v)
            d = (m >> 24) + 128
            plsc.addupdate_scatter(hist, [lanes * 256 + d], ones)
            return c

        lax.fori_loop(0, NVEC, hist0_body, 0)
        _merge_hist(hist, tot)
        d0, kp = _find_boundary(tot, jnp.int32(K))

        def comp0_body(i, cnt):
            v = rowbuf[pl.ds(pl.multiple_of(i * 16, 16), 16)]
            m = _monotone(v)
            sel = ((m >> 24) + 128) == d0
            pc = _popcount_scalar(sel)

            def insert(cnt):
                seli = sel.astype(jnp.int32)
                idx = cnt + plsc.cumsum(seli) - seli
                plsc.store_scatter(cand, [idx], m, mask=sel)
                return cnt + pc

            return lax.cond(pc > 0, insert, lambda c: c, cnt)

        cnt = lax.fori_loop(0, NVEC, comp0_body, jnp.int32(0))

        digits = [d0]
        for shift in (16, 8, 0):
            _zero_hist(hist)
            nv = (cnt + 15) >> 4

            def histl_body(i, c, shift=shift):
                mk = cand[pl.ds(pl.multiple_of(i * 16, 16), 16)]
                valid = (i * 16 + lanes) < cnt
                d = (mk >> shift) & 255
                plsc.addupdate_scatter(hist, [lanes * 256 + d], ones, mask=valid)
                return c

            lax.fori_loop(0, nv, histl_body, 0)
            _merge_hist(hist, tot)
            dl, kp = _find_boundary(tot, kp)
            digits.append(dl)

            if shift > 0:
                def compl_body(i, cnt2, shift=shift, dl=dl, cnt=cnt):
                    mk = cand[pl.ds(pl.multiple_of(i * 16, 16), 16)]
                    valid = (i * 16 + lanes) < cnt
                    sel = valid & (((mk >> shift) & 255) == dl)
                    pc = _popcount_scalar(sel)

                    def insert(c2):
                        seli = sel.astype(jnp.int32)
                        idx = c2 + plsc.cumsum(seli) - seli
                        plsc.store_scatter(cand, [idx], mk, mask=sel)
                        return c2 + pc

                    return lax.cond(pc > 0, insert, lambda c2: c2, cnt2)

                cnt = lax.fori_loop(0, nv, compl_body, jnp.int32(0))

        d0s, d1, d2, d3 = digits
        m_t = ((d0s - 128) << 24) | (d1 << 16) | (d2 << 8) | d3

        m_tv = jnp.full((16,), 0, jnp.int32) + m_t
        b_tv = jnp.where(
            m_tv >= 0,
            m_tv,
            jnp.bitwise_not(jnp.bitwise_xor(m_tv, jnp.int32(_INT_MIN))),
        )
        t_v = lax.bitcast_convert_type(b_tv, jnp.float32)

        def mask_body(i, c):
            off = pl.multiple_of(i * 16, 16)
            v = rowbuf[pl.ds(off, 16)]
            keep = (v >= t_v) & (v > 0.0)
            rowbuf[pl.ds(off, 16)] = jnp.where(keep, v, 0.0)
            return c

        lax.fori_loop(0, NVEC, mask_body, 0)
        pltpu.sync_copy(rowbuf, out_hbm.at[row])
        return carry

    lax.fori_loop(0, ROWS_PER_W, do_row, 0)


@jax.jit
def kernel(x):
    mesh = plsc.VectorSubcoreMesh(core_axis_name="c", subcore_axis_name="s")
    f = functools.partial(
        pl.kernel,
        mesh=mesh,
        out_type=jax.ShapeDtypeStruct((ROWS, COLS), jnp.float32),
        scratch_types=[
            pltpu.VMEM((COLS,), jnp.float32),
            pltpu.VMEM((COLS,), jnp.int32),
            pltpu.VMEM((16 * 256,), jnp.int32),
            pltpu.VMEM((256,), jnp.int32),
        ],
        compiler_params=pltpu.CompilerParams(needs_layout_passes=False),
    )(_sc_body)
    return f(x)


# speedup vs baseline: 4.6296x; 4.6296x over previous
import functools

import jax
import jax.numpy as jnp
from jax import lax
from jax.experimental import pallas as pl
from jax.experimental.pallas import tpu as pltpu
from jax.experimental.pallas import tpu_sc as plsc

K = 64
ROWS = 128
COLS = 32768
NVEC = COLS // 16
NC = 2
NS = 16
NW = NC * NS
ROWS_PER_W = ROWS // NW

_INT_MIN = -(2 ** 31)


def _lanes():
    return jnp.arange(16, dtype=jnp.int32)


def _splat_to_scalar(v):
    return lax.reduce_max(v, axes=(0,))


def _extract(v, idx):
    return lax.reduce_max(
        jnp.where(_lanes() == idx, v, jnp.int32(_INT_MIN)), axes=(0,)
    )


def _popcount_splat(mask):
    return plsc.all_reduce_population_count(mask)


def _monotone(v):
    b = lax.bitcast_convert_type(v, jnp.int32)
    return jnp.where(
        b >= 0, b, jnp.bitwise_xor(jnp.bitwise_not(b), jnp.int32(_INT_MIN))
    )


def _zero_hist(hist):
    @plsc.parallel_loop(0, 256, unroll=8)
    def _(j):
        hist[pl.ds(pl.multiple_of(j * 16, 16), 16)] = jnp.zeros(16, jnp.int32)


def _merge_hist(hist, tot):
    @plsc.parallel_loop(0, 16, unroll=2)
    def _(j):
        off = pl.multiple_of(j * 16, 16)
        acc = hist[pl.ds(off, 16)]
        for l in range(1, 16):
            acc = acc + hist[pl.ds(l * 256 + off, 16)]
        tot[pl.ds(off, 16)] = acc


def _find_boundary(tot, kp):
    lanes = _lanes()
    s = jnp.zeros(16, jnp.int32)
    for j in range(16):
        sj = lax.reduce_sum(tot[pl.ds(j * 16, 16)], axes=(0,))
        s = jnp.where(lanes == j, sj, s)
    rev_s = lax.rev(s, (0,))
    cs_s = plsc.cumsum(rev_s)
    hit_s = cs_s >= kp
    lc = jnp.int32(16) - _splat_to_scalar(_popcount_splat(hit_s))
    jc = jnp.int32(15) - lc
    above_chunks = _extract(cs_s, lc) - _extract(rev_s, lc)

    chunk = tot[pl.ds(pl.multiple_of(jc * 16, 16), 16)]
    rchunk = lax.rev(chunk, (0,))
    cs2 = above_chunks + plsc.cumsum(rchunk)
    hit2 = cs2 >= kp
    l2 = jnp.int32(16) - _splat_to_scalar(_popcount_splat(hit2))
    dstar = jc * 16 + jnp.int32(15) - l2
    cnt_gt = _extract(cs2, l2) - _extract(rchunk, l2)
    return dstar, kp - cnt_gt


def _sc_body(x_hbm, out_hbm, rowbuf, cand, cand2, hist, tot):
    wid = lax.axis_index("s") * NC + lax.axis_index("c")
    lanes = _lanes()
    ones = jnp.ones(16, jnp.int32)
    base = lanes * 256

    def do_row(r, carry):
        row = wid * ROWS_PER_W + r
        pltpu.sync_copy(x_hbm.at[row], rowbuf)

        _zero_hist(hist)

        @plsc.parallel_loop(0, NVEC, unroll=8)
        def _(i):
            v = rowbuf[pl.ds(pl.multiple_of(i * 16, 16), 16)]
            m = _monotone(v)
            d = (m >> 24) + 128
            plsc.addupdate_scatter(hist, [base + d], ones)

        _merge_hist(hist, tot)
        d0, kp = _find_boundary(tot, jnp.int32(K))

        @plsc.parallel_loop(0, NVEC, unroll=8, carry=jnp.zeros(16, jnp.int32))
        def cnt_vec(i, cv):
            v = rowbuf[pl.ds(pl.multiple_of(i * 16, 16), 16)]
            m = _monotone(v)
            sel = ((m >> 24) + 128) == d0
            seli = sel.astype(jnp.int32)
            idx = cv + plsc.cumsum(seli) - seli
            plsc.store_scatter(cand, [idx], m, mask=sel)
            return cv + _popcount_splat(sel)

        cnt = _splat_to_scalar(cnt_vec)

        digits = [d0]
        src, dst = cand, cand2
        for shift in (16, 8, 0):
            _zero_hist(hist)
            nv = (cnt + 15) >> 4

            @plsc.parallel_loop(0, nv, unroll=2)
            def _(i, src=src, shift=shift, cnt=cnt):
                mk = src[pl.ds(pl.multiple_of(i * 16, 16), 16)]
                valid = (i * 16 + lanes) < cnt
                d = (mk >> shift) & 255
                plsc.addupdate_scatter(hist, [base + d], ones, mask=valid)

            _merge_hist(hist, tot)
            dl, kp = _find_boundary(tot, kp)
            digits.append(dl)

            if shift > 0:
                @plsc.parallel_loop(
                    0, nv, unroll=2, carry=jnp.zeros(16, jnp.int32)
                )
                def cv2(i, cv, src=src, dst=dst, shift=shift, dl=dl, cnt=cnt):
                    mk = src[pl.ds(pl.multiple_of(i * 16, 16), 16)]
                    valid = (i * 16 + lanes) < cnt
                    sel = valid & (((mk >> shift) & 255) == dl)
                    seli = sel.astype(jnp.int32)
                    idx = cv + plsc.cumsum(seli) - seli
                    plsc.store_scatter(dst, [idx], mk, mask=sel)
                    return cv + _popcount_splat(sel)

                cnt = _splat_to_scalar(cv2)
                src, dst = dst, src

        d0s, d1, d2, d3 = digits
        m_t = ((d0s - 128) << 24) | (d1 << 16) | (d2 << 8) | d3

        m_tv = jnp.zeros(16, jnp.int32) + m_t
        b_tv = jnp.where(
            m_tv >= 0,
            m_tv,
            jnp.bitwise_not(jnp.bitwise_xor(m_tv, jnp.int32(_INT_MIN))),
        )
        t_v = lax.bitcast_convert_type(b_tv, jnp.float32)

        @plsc.parallel_loop(0, NVEC, unroll=8)
        def _(i):
            off = pl.multiple_of(i * 16, 16)
            v = rowbuf[pl.ds(off, 16)]
            keep = (v >= t_v) & (v > 0.0)
            rowbuf[pl.ds(off, 16)] = jnp.where(keep, v, 0.0)

        pltpu.sync_copy(rowbuf, out_hbm.at[row])
        return carry

    lax.fori_loop(0, ROWS_PER_W, do_row, 0)


@jax.jit
def kernel(x):
    mesh = plsc.VectorSubcoreMesh(core_axis_name="c", subcore_axis_name="s")
    f = pl.kernel(
        _sc_body,
        mesh=mesh,
        out_type=jax.ShapeDtypeStruct((ROWS, COLS), jnp.float32),
        scratch_types=[
            pltpu.VMEM((COLS,), jnp.float32),
            pltpu.VMEM((COLS,), jnp.int32),
            pltpu.VMEM((COLS,), jnp.int32),
            pltpu.VMEM((16 * 256,), jnp.int32),
            pltpu.VMEM((256,), jnp.int32),
        ],
        compiler_params=pltpu.CompilerParams(needs_layout_passes=False),
    )
    return f(x)
